# baseline (speedup 1.0000x reference)
import functools

import jax
import jax.numpy as jnp
from jax.experimental import pallas as pl
from jax.experimental.pallas import tpu as pltpu


def _gcn_body(dlf_ref, qm_ref, spk_ref, w1_ref, b1_ref, cw_ref, cb_ref,
              l_hbm, a_hbm, v_hbm, out_ref,
              lb, ab, vb, stage, lsem, ssem, *, n_dia, dia, num_k, nch):
    total = n_dia * dia
    d = lb.shape[1]
    f32 = jnp.float32
    rows = total // nch
    grp = n_dia // nch

    for c in range(nch):
        sl = pl.ds(c * rows, rows)
        pltpu.make_async_copy(l_hbm.at[sl, :], lb.at[sl, :], lsem.at[0, c]).start()
        pltpu.make_async_copy(a_hbm.at[sl, :], ab.at[sl, :], lsem.at[1, c]).start()
        pltpu.make_async_copy(v_hbm.at[sl, :], vb.at[sl, :], lsem.at[2, c]).start()

    scale = jnp.sum(dlf_ref[0, :]) / f32(dlf_ref.shape[1] * dia)
    spk_add = jnp.dot(qm_ref[...], spk_ref[...], preferred_element_type=f32)
    w1t = w1_ref[...].T
    b1 = b1_ref[...]
    inv = f32(1.0 / (dia + 2))
    wkts = [cw_ref[k].T * inv for k in range(num_k)]

    for c in range(nch):
        sl = pl.ds(c * rows, rows)
        pltpu.make_async_copy(l_hbm.at[sl, :], lb.at[sl, :], lsem.at[0, c]).wait()
        pltpu.make_async_copy(a_hbm.at[sl, :], ab.at[sl, :], lsem.at[1, c]).wait()
        pltpu.make_async_copy(v_hbm.at[sl, :], vb.at[sl, :], lsem.at[2, c]).wait()

        xl = (lb[sl, :] + spk_add[c * rows:(c + 1) * rows, :]) * scale
        xa = ab[sl, :] * scale
        xv = vb[sl, :] * scale
        hl = jnp.dot(xl, w1t, preferred_element_type=f32) + b1
        ha = jnp.dot(xa, w1t, preferred_element_type=f32) + b1
        hv = jnp.dot(xv, w1t, preferred_element_type=f32) + b1

        gl, ga, gv = hl, ha, hv
        for k in range(num_k):
            bk = cb_ref[k:k + 1, :]
            outs = []
            for g, o1, o2 in ((gl, ga, gv), (ga, gl, gv), (gv, gl, ga)):
                s = jnp.sum(g.reshape(grp, dia, d), axis=1, keepdims=True)
                sb = jnp.broadcast_to(s, (grp, dia, d)).reshape(rows, d)
                agg = sb + (o1 + o2)
                outs.append(
                    g + jnp.dot(agg, wkts[k], preferred_element_type=f32) + bk)
            gl, ga, gv = outs

        for col, val in enumerate((xl, hl, gl, xa, ha, ga, xv, hv, gv)):
            stage[sl, col * d:(col + 1) * d] = val
        pltpu.make_async_copy(stage.at[sl, :], out_ref.at[sl, :],
                              ssem.at[c]).start()

    for c in range(nch):
        sl = pl.ds(c * rows, rows)
        pltpu.make_async_copy(stage.at[sl, :], out_ref.at[sl, :],
                              ssem.at[c]).wait()


def kernel(a, v, l, qmask, dia_len, epoch, spk_emb, fc1_w, fc1_b, conv_w,
           conv_b):
    del epoch
    total, d = a.shape
    n_dia = dia_len.shape[0]
    dia = qmask.shape[0]
    num_k = conv_w.shape[0]
    nspk = qmask.shape[2]

    nch = 5
    while n_dia % nch:
        nch -= 1

    qm = jnp.transpose(qmask, (1, 0, 2)).reshape(total, nspk)
    dlf = dia_len.astype(jnp.float32).reshape(1, n_dia)
    b1 = fc1_b.reshape(1, -1)

    body = functools.partial(_gcn_body, n_dia=n_dia, dia=dia, num_k=num_k,
                             nch=nch)
    hbm = pl.BlockSpec(memory_space=pltpu.MemorySpace.HBM)
    out = pl.pallas_call(
        body,
        in_specs=[
            pl.BlockSpec((1, n_dia), lambda: (0, 0)),
            pl.BlockSpec((total, nspk), lambda: (0, 0)),
            pl.BlockSpec((spk_emb.shape[0], d), lambda: (0, 0)),
            pl.BlockSpec((d, d), lambda: (0, 0)),
            pl.BlockSpec((1, d), lambda: (0, 0)),
            pl.BlockSpec((num_k, d, d), lambda: (0, 0, 0)),
            pl.BlockSpec((num_k, d), lambda: (0, 0)),
            hbm,
            hbm,
            hbm,
        ],
        out_specs=hbm,
        out_shape=jax.ShapeDtypeStruct((total, 9 * d), jnp.float32),
        scratch_shapes=[
            pltpu.VMEM((total, d), jnp.float32),
            pltpu.VMEM((total, d), jnp.float32),
            pltpu.VMEM((total, d), jnp.float32),
            pltpu.VMEM((total, 9 * d), jnp.float32),
            pltpu.SemaphoreType.DMA((3, nch)),
            pltpu.SemaphoreType.DMA((nch,)),
        ],
    )(dlf, qm, spk_emb, fc1_w, b1, conv_w, conv_b, l, a, v)
    return out


# speedup vs baseline: 1.0808x; 1.0808x over previous
import functools

import jax
import jax.numpy as jnp
from jax.experimental import pallas as pl
from jax.experimental.pallas import tpu as pltpu


def _eye(n, dtype):
    r = jax.lax.broadcasted_iota(jnp.int32, (n, n), 0)
    c = jax.lax.broadcasted_iota(jnp.int32, (n, n), 1)
    return (r == c).astype(dtype)


def _gcn_body(dlf_ref, qm_ref, l_ref, a_ref, v_ref, spk_ref, w1_ref, b1_ref,
              cw_ref, cb_ref, out_ref, stage, u0, sems, *, n_dia, dia, num_k):
    total = n_dia * dia
    d = l_ref.shape[1]
    f32 = jnp.float32

    def put(col, val):
        stage[:, col * d:(col + 1) * d] = val
        pltpu.make_async_copy(stage.at[:, col * d:(col + 1) * d],
                              out_ref.at[:, col * d:(col + 1) * d],
                              sems.at[col]).start()

    def wait(col):
        pltpu.make_async_copy(stage.at[:, col * d:(col + 1) * d],
                              out_ref.at[:, col * d:(col + 1) * d],
                              sems.at[col]).wait()

    def dot(x, y):
        return jnp.dot(x, y, preferred_element_type=f32)

    def seg_sum(g):
        return jnp.sum(g.reshape(n_dia, dia, d), axis=1)

    scale = jnp.sum(dlf_ref[0, :]) / f32(dlf_ref.shape[1] * dia)

    xl = (l_ref[...] + dot(qm_ref[...], spk_ref[...])) * scale
    xa = a_ref[...] * scale
    xv = v_ref[...] * scale
    put(0, xl)
    put(3, xa)
    put(6, xv)

    w1t = w1_ref[...].T
    b1 = b1_ref[...]
    hl = dot(xl, w1t) + b1
    ha = dot(xa, w1t) + b1
    hv = dot(xv, w1t) + b1
    put(1, hl)
    put(4, ha)
    put(7, hv)
    u0[:, 0 * d:1 * d] = hl
    u0[:, 1 * d:2 * d] = ha
    u0[:, 2 * d:3 * d] = hv

    U = jnp.concatenate([seg_sum(hl), seg_sum(ha), seg_sum(hv)], axis=1)

    ident = _eye(d, f32)
    inv = f32(1.0 / (dia + 2))
    fdia = f32(dia)
    P = None
    D = None
    for k in range(num_k):
        w = cw_ref[k].T * inv
        bk = cb_ref[k:k + 1, :]
        b3 = jnp.concatenate([bk, bk, bk], axis=1)
        wd = ident + fdia * w
        A = jnp.concatenate([
            jnp.concatenate([ident, w, w], axis=1),
            jnp.concatenate([w, ident, w], axis=1),
            jnp.concatenate([w, w, ident], axis=1)], axis=0)
        C = jnp.concatenate([
            jnp.concatenate([wd, w, w], axis=1),
            jnp.concatenate([w, wd, w], axis=1),
            jnp.concatenate([w, w, wd], axis=1)], axis=0)
        UB = jnp.concatenate([dot(U[:, 0 * d:1 * d], w),
                              dot(U[:, 1 * d:2 * d], w),
                              dot(U[:, 2 * d:3 * d], w)], axis=1) + b3
        if k == 0:
            P = A
            D = UB
        else:
            P = dot(P, A)
            D = dot(D, A) + UB
        if k + 1 < num_k:
            U = dot(U, C) + fdia * b3

    g = dot(u0[...], P) + jnp.broadcast_to(
        D.reshape(n_dia, 1, 3 * d), (n_dia, dia, 3 * d)).reshape(total, 3 * d)

    put(2, g[:, 0 * d:1 * d])
    put(5, g[:, 1 * d:2 * d])
    put(8, g[:, 2 * d:3 * d])
    for col in range(9):
        wait(col)


def kernel(a, v, l, qmask, dia_len, epoch, spk_emb, fc1_w, fc1_b, conv_w,
           conv_b):
    del epoch
    total, d = a.shape
    n_dia = dia_len.shape[0]
    dia = qmask.shape[0]
    num_k = conv_w.shape[0]
    nspk = qmask.shape[2]

    qm = jnp.transpose(qmask, (1, 0, 2)).reshape(total, nspk)
    dlf = dia_len.astype(jnp.float32).reshape(1, n_dia)
    b1 = fc1_b.reshape(1, -1)

    body = functools.partial(_gcn_body, n_dia=n_dia, dia=dia, num_k=num_k)
    out = pl.pallas_call(
        body,
        in_specs=[
            pl.BlockSpec((1, n_dia), lambda: (0, 0)),
            pl.BlockSpec((total, nspk), lambda: (0, 0)),
            pl.BlockSpec((total, d), lambda: (0, 0)),
            pl.BlockSpec((total, d), lambda: (0, 0)),
            pl.BlockSpec((total, d), lambda: (0, 0)),
            pl.BlockSpec((spk_emb.shape[0], d), lambda: (0, 0)),
            pl.BlockSpec((d, d), lambda: (0, 0)),
            pl.BlockSpec((1, d), lambda: (0, 0)),
            pl.BlockSpec((num_k, d, d), lambda: (0, 0, 0)),
            pl.BlockSpec((num_k, d), lambda: (0, 0)),
        ],
        out_specs=pl.BlockSpec(memory_space=pltpu.MemorySpace.HBM),
        out_shape=jax.ShapeDtypeStruct((total, 9 * d), jnp.float32),
        scratch_shapes=[
            pltpu.VMEM((total, 9 * d), jnp.float32),
            pltpu.VMEM((total, 3 * d), jnp.float32),
            pltpu.SemaphoreType.DMA((9,)),
        ],
    )(dlf, qm, l, a, v, spk_emb, fc1_w, b1, conv_w, conv_b)
    return out


# speedup vs baseline: 1.1974x; 1.1079x over previous
import functools

import jax
import jax.numpy as jnp
from jax.experimental import pallas as pl
from jax.experimental.pallas import tpu as pltpu


def _eye(n, dtype):
    r = jax.lax.broadcasted_iota(jnp.int32, (n, n), 0)
    c = jax.lax.broadcasted_iota(jnp.int32, (n, n), 1)
    return (r == c).astype(dtype)


def _gcn_body(dlf_ref, qm_ref, spk_ref, w1_ref, b1_ref, cw_ref, cb_ref,
              l_hbm, a_hbm, v_hbm, out_ref,
              lb, ab, vb, stage, lsem, ssem, *, n_dia, dia, num_k, nch):
    total = n_dia * dia
    d = lb.shape[1]
    f32 = jnp.float32
    rows = total // nch
    grp = n_dia // nch

    def dot(x, y):
        return jnp.dot(x, y, preferred_element_type=f32)

    def pmul(p1, p2):
        m1, n1 = p1
        m2, n2 = p2
        return (dot(m1, m2), dot(m1, n2) + dot(n1, m2) + 3.0 * dot(n1, n2))

    for c in range(nch):
        sl = pl.ds(c * rows, rows)
        pltpu.make_async_copy(l_hbm.at[sl, :], lb.at[sl, :], lsem.at[0, c]).start()
        pltpu.make_async_copy(a_hbm.at[sl, :], ab.at[sl, :], lsem.at[1, c]).start()
        pltpu.make_async_copy(v_hbm.at[sl, :], vb.at[sl, :], lsem.at[2, c]).start()

    scale = jnp.sum(dlf_ref[0, :]) / f32(dlf_ref.shape[1] * dia)
    spk_add = dot(qm_ref[...], spk_ref[...])
    w1t = w1_ref[...].T
    b1 = b1_ref[...]

    ident = _eye(d, f32)
    inv = f32(1.0 / (dia + 2))
    fdia = f32(dia)

    w0 = cw_ref[0].T * inv
    b0 = cb_ref[0:1, :]
    P = (ident - w0, w0)
    Q = (w0, jnp.zeros_like(w0))
    R = (ident + (fdia - 1.0) * w0, w0)
    rho = b0
    sig = fdia * b0
    for k in range(1, num_k):
        wk = cw_ref[k].T * inv
        bk = cb_ref[k:k + 1, :]
        A = (ident - wk, wk)
        P = pmul(P, A)
        Q = tuple(x + y for x, y in zip(pmul(Q, A),
                                        (dot(R[0], wk), dot(R[1], wk))))
        new_rho = dot(rho, ident + 2.0 * wk) + dot(sig, wk) + bk
        sig = dot(sig, ident + (fdia + 2.0) * wk) + fdia * bk
        rho = new_rho
        if k + 1 < num_k:
            R = pmul(R, (ident + (fdia - 1.0) * wk, wk))
    MP, NP = P
    MQ, NQ = Q

    for c in range(nch):
        sl = pl.ds(c * rows, rows)
        pltpu.make_async_copy(l_hbm.at[sl, :], lb.at[sl, :], lsem.at[0, c]).wait()
        pltpu.make_async_copy(a_hbm.at[sl, :], ab.at[sl, :], lsem.at[1, c]).wait()
        pltpu.make_async_copy(v_hbm.at[sl, :], vb.at[sl, :], lsem.at[2, c]).wait()

        xl = (lb[sl, :] + spk_add[c * rows:(c + 1) * rows, :]) * scale
        xa = ab[sl, :] * scale
        xv = vb[sl, :] * scale
        hl = dot(xl, w1t) + b1
        ha = dot(xa, w1t) + b1
        hv = dot(xv, w1t) + b1

        rowtot = hl + ha + hv
        ul = jnp.sum(hl.reshape(grp, dia, d), axis=1)
        ua = jnp.sum(ha.reshape(grp, dia, d), axis=1)
        uv = jnp.sum(hv.reshape(grp, dia, d), axis=1)
        utot = ul + ua + uv
        rt_np = dot(rowtot, NP)
        ut_nq = dot(utot, NQ) + rho

        gs = []
        for hm, um in ((hl, ul), (ha, ua), (hv, uv)):
            dm = dot(um, MQ) + ut_nq
            db = jnp.broadcast_to(dm.reshape(grp, 1, d),
                                  (grp, dia, d)).reshape(rows, d)
            gs.append(dot(hm, MP) + rt_np + db)

        for col, val in enumerate((xl, hl, gs[0], xa, ha, gs[1],
                                   xv, hv, gs[2])):
            stage[sl, col * d:(col + 1) * d] = val
        pltpu.make_async_copy(stage.at[sl, :], out_ref.at[sl, :],
                              ssem.at[c]).start()

    for c in range(nch):
        sl = pl.ds(c * rows, rows)
        pltpu.make_async_copy(stage.at[sl, :], out_ref.at[sl, :],
                              ssem.at[c]).wait()


def kernel(a, v, l, qmask, dia_len, epoch, spk_emb, fc1_w, fc1_b, conv_w,
           conv_b):
    del epoch
    total, d = a.shape
    n_dia = dia_len.shape[0]
    dia = qmask.shape[0]
    num_k = conv_w.shape[0]
    nspk = qmask.shape[2]

    nch = 5
    while n_dia % nch:
        nch -= 1

    qm = jnp.transpose(qmask, (1, 0, 2)).reshape(total, nspk)
    dlf = dia_len.astype(jnp.float32).reshape(1, n_dia)
    b1 = fc1_b.reshape(1, -1)

    body = functools.partial(_gcn_body, n_dia=n_dia, dia=dia, num_k=num_k,
                             nch=nch)
    hbm = pl.BlockSpec(memory_space=pltpu.MemorySpace.HBM)
    out = pl.pallas_call(
        body,
        in_specs=[
            pl.BlockSpec((1, n_dia), lambda: (0, 0)),
            pl.BlockSpec((total, nspk), lambda: (0, 0)),
            pl.BlockSpec((spk_emb.shape[0], d), lambda: (0, 0)),
            pl.BlockSpec((d, d), lambda: (0, 0)),
            pl.BlockSpec((1, d), lambda: (0, 0)),
            pl.BlockSpec((num_k, d, d), lambda: (0, 0, 0)),
            pl.BlockSpec((num_k, d), lambda: (0, 0)),
            hbm,
            hbm,
            hbm,
        ],
        out_specs=hbm,
        out_shape=jax.ShapeDtypeStruct((total, 9 * d), jnp.float32),
        scratch_shapes=[
            pltpu.VMEM((total, d), jnp.float32),
            pltpu.VMEM((total, d), jnp.float32),
            pltpu.VMEM((total, d), jnp.float32),
            pltpu.VMEM((total, 9 * d), jnp.float32),
            pltpu.SemaphoreType.DMA((3, nch)),
            pltpu.SemaphoreType.DMA((nch,)),
        ],
    )(dlf, qm, spk_emb, fc1_w, b1, conv_w, conv_b, l, a, v)
    return out
